# baseline (speedup 1.0000x reference)
import jax
import jax.numpy as jnp
from jax import lax
from jax.experimental import pallas as pl
from jax.experimental.pallas import tpu as pltpu
from jax.experimental.pallas import tpu_sc as plsc

N = 10000
E = 320000
G = 64

NC = 2
NS = 16
CH = 96
EPT = 20160
E_PAD = NS * EPT
NROWS = N + 8
ROWS = 624
TAIL = N - ROWS * NS

ROWB = 1000
GRID = N // ROWB


def _sc_mesh():
    return plsc.VectorSubcoreMesh(core_axis_name="c", subcore_axis_name="s")


def _zero_rows(ref, nrows, width):
    z16 = jnp.zeros((16,), jnp.float32)

    def zrow(r, carry):
        for j in range(width // 16):
            ref[r, pl.ds(j * 16, 16)] = z16
        return carry

    lax.fori_loop(0, nrows, zrow, 0)


def _init_table(bounce, acc_sh, s, width):
    _zero_rows(bounce, ROWS + TAIL, width)
    row0 = pl.multiple_of(s * ROWS, 8)
    pltpu.sync_copy(bounce.at[pl.ds(0, ROWS)], acc_sh.at[pl.ds(row0, ROWS)])

    @pl.when(s == NS - 1)
    def _():
        pltpu.sync_copy(bounce.at[pl.ds(ROWS, TAIL)],
                        acc_sh.at[pl.ds(NS * ROWS, TAIL)])


def _drain_table(bounce, acc_sh, out_h, c, s):
    row0 = pl.multiple_of(s * ROWS, 8)
    pltpu.sync_copy(acc_sh.at[pl.ds(row0, ROWS)], bounce.at[pl.ds(0, ROWS)])
    pltpu.sync_copy(bounce.at[pl.ds(0, ROWS)], out_h.at[c, pl.ds(row0, ROWS)])

    @pl.when(s == NS - 1)
    def _():
        pltpu.sync_copy(acc_sh.at[pl.ds(NS * ROWS, TAIL)],
                        bounce.at[pl.ds(ROWS, TAIL)])
        pltpu.sync_copy(bounce.at[pl.ds(ROWS, TAIL)],
                        out_h.at[c, pl.ds(NS * ROWS, TAIL)])


NBUF = 10

_SLICES = [(0, 96), (96, 96), (192, 96), (288, 96), (384, 96),
           (480, 96), (576, 48)]


def _init_table_slots(rows, acc_sh, s, width, nbuf):
    _zero_rows(rows.at[0], CH, width)
    row0 = pl.multiple_of(s * ROWS, 8)
    for off, sz in _SLICES:
        pltpu.sync_copy(rows.at[0, pl.ds(0, sz)],
                        acc_sh.at[pl.ds(row0 + off, sz)])

    @pl.when(s == NS - 1)
    def _():
        pltpu.sync_copy(rows.at[0, pl.ds(0, TAIL)],
                        acc_sh.at[pl.ds(NS * ROWS, TAIL)])


def _drain_table_slots(rows, acc_sh, out_h, c, s, nbuf, isem, ssem):
    row0 = pl.multiple_of(s * ROWS, 8)
    outd = {}
    for j, (off, sz) in enumerate(_SLICES):
        b = j % nbuf
        if b in outd:
            outd[b].wait()
        d1 = pltpu.async_copy(acc_sh.at[pl.ds(row0 + off, sz)],
                              rows.at[b, pl.ds(0, sz)], isem.at[b])
        d1.wait()
        outd[b] = pltpu.async_copy(rows.at[b, pl.ds(0, sz)],
                                   out_h.at[c, pl.ds(row0 + off, sz)],
                                   ssem.at[b])
    for b, d in outd.items():
        d.wait()

    @pl.when(s == NS - 1)
    def _():
        t0 = NS * ROWS
        d1 = pltpu.async_copy(acc_sh.at[pl.ds(t0, TAIL)],
                              rows.at[0, pl.ds(0, TAIL)], isem.at[0])
        d1.wait()
        d2 = pltpu.async_copy(rows.at[0, pl.ds(0, TAIL)],
                              out_h.at[c, pl.ds(t0, TAIL)], ssem.at[0])
        d2.wait()


def _sc_scatter(ei, zs0, zs1, doh, nbuf):
    ept = EPT
    group = nbuf * CH
    ngroups = ept // group
    assert ngroups * group == ept

    def body(ei_h, zs0_h, zs1_h, out_h, eidx, rows, acc_sh,
             isem, gsem, ssem):
        c = lax.axis_index("c")
        s = lax.axis_index("s")
        base = s * ept

        def phase(zs_h):
            idesc0 = []
            for b in range(nbuf):
                off = pl.multiple_of(base + b * CH, 16)
                idesc0.append(pltpu.async_copy(
                    ei_h.at[:, pl.ds(off, CH)], eidx.at[b], isem.at[b]))
            _init_table_slots(rows, acc_sh, s, doh, nbuf)
            gdesc0 = []
            for b in range(nbuf):
                idesc0[b].wait()
                gdesc0.append(pltpu.async_copy(
                    zs_h.at[eidx.at[b, 0]], rows.at[b], gsem.at[b]))
            plsc.subcore_barrier()
            for b in range(nbuf):
                gdesc0[b].wait()
                pltpu.async_copy(
                    rows.at[b], acc_sh.at[eidx.at[b, 1]], ssem.at[b],
                    add=True)

            def inner(g, pg):
                goff = pl.multiple_of(base + g * group, 16)
                idesc = []
                for b in range(nbuf):
                    idesc.append(pltpu.async_copy(
                        ei_h.at[:, pl.ds(goff + b * CH, CH)],
                        eidx.at[pg + b], isem.at[b]))
                gdesc = []
                for b in range(nbuf):
                    idesc[b].wait()
                    pltpu.make_async_copy(
                        zs_h.at[pl.ds(0, CH)], rows.at[b],
                        ssem.at[b]).wait()
                    gdesc.append(pltpu.async_copy(
                        zs_h.at[eidx.at[pg + b, 0]], rows.at[b], gsem.at[b]))
                for b in range(nbuf):
                    gdesc[b].wait()
                    pltpu.async_copy(
                        rows.at[b], acc_sh.at[eidx.at[pg + b, 1]], ssem.at[b],
                        add=True)

            def gbody(g, carry):
                @pl.when(g % 2 == 1)
                def _():
                    inner(g, nbuf)

                @pl.when(g % 2 == 0)
                def _():
                    inner(g, 0)

                return carry

            lax.fori_loop(1, ngroups, gbody, 0)
            for b in range(nbuf):
                pltpu.make_async_copy(
                    zs_h.at[pl.ds(0, CH)], rows.at[b], ssem.at[b]).wait()

        @pl.when(c == 0)
        def _():
            phase(zs0_h)

        @pl.when(c == 1)
        def _():
            phase(zs1_h)

        plsc.subcore_barrier()
        _drain_table_slots(rows, acc_sh, out_h, c, s, nbuf, isem, ssem)

    scratch = [
        pltpu.VMEM((2 * nbuf, 2, CH), jnp.int32),
        pltpu.VMEM((nbuf, CH, doh), jnp.float32),
        pltpu.VMEM_SHARED((NROWS, doh), jnp.float32),
        pltpu.SemaphoreType.DMA((nbuf,)),
        pltpu.SemaphoreType.DMA((nbuf,)),
        pltpu.SemaphoreType.DMA((nbuf,)),
    ]
    return pl.kernel(
        body,
        out_type=jax.ShapeDtypeStruct((NC, N, doh), jnp.float32),
        mesh=_sc_mesh(),
        scratch_types=scratch,
        compiler_params=pltpu.CompilerParams(use_tc_tiling_on_sc=False),
    )(ei, zs0, zs1)


def _sc_degree(ei):
    W = 16
    ept = EPT // NC
    nbuf = 7
    group = nbuf * CH
    ngroups = ept // group

    def body(ei_h, out_h, didx, ones_v, bounce, acc_sh, isem, ssem):
        c = lax.axis_index("c")
        s = lax.axis_index("s")
        one16 = jnp.ones((16,), jnp.float32)

        def orow(r, carry):
            ones_v[r, pl.ds(0, 16)] = one16
            return carry

        lax.fori_loop(0, CH, orow, 0)
        _init_table(bounce, acc_sh, s, W)
        plsc.subcore_barrier()

        base = c * (E_PAD // NC) + s * ept

        def gbody(g, carry):
            goff = pl.multiple_of(base + g * group, 16)
            idesc = []
            for b in range(nbuf):
                idesc.append(pltpu.async_copy(
                    ei_h.at[1, pl.ds(goff + b * CH, CH)],
                    didx.at[b], isem.at[b]))
            sdesc = []
            for b in range(nbuf):
                idesc[b].wait()
                sdesc.append(pltpu.async_copy(
                    ones_v, acc_sh.at[didx.at[b]], ssem.at[b], add=True))
            for b in range(nbuf):
                sdesc[b].wait()
            return carry

        lax.fori_loop(0, ngroups, gbody, 0)
        plsc.subcore_barrier()
        _drain_table(bounce, acc_sh, out_h, c, s)

    scratch = [
        pltpu.VMEM((7, CH), jnp.int32),
        pltpu.VMEM((CH, W), jnp.float32),
        pltpu.VMEM((ROWS + TAIL, W), jnp.float32),
        pltpu.VMEM_SHARED((NROWS, W), jnp.float32),
        pltpu.SemaphoreType.DMA((7,)),
        pltpu.SemaphoreType.DMA((7,)),
    ]
    return pl.kernel(
        body,
        out_type=jax.ShapeDtypeStruct((NC, N, W), jnp.float32),
        mesh=_sc_mesh(),
        scratch_types=scratch,
        compiler_params=pltpu.CompilerParams(use_tc_tiling_on_sc=False),
    )(ei)


def _lrelu(x):
    return jnp.where(x > 0, x, 0.01 * x)


def _dinv_block(degp_ref):
    deg = degp_ref[0, :, 0:1] + degp_ref[1, :, 0:1] + 1.0
    return lax.rsqrt(deg)


def _split_specs(do):
    doh = do // 2
    outs = [pl.BlockSpec((ROWB, doh), lambda i: (i, 0)) for _ in range(2)]
    shapes = [jax.ShapeDtypeStruct((N, doh), jnp.float32) for _ in range(2)]
    return doh, outs, shapes


def _tc_first(x, W1, degp):
    d_in, do = W1.shape
    doh, outs, shapes = _split_specs(do)

    def body(x_ref, w_ref, degp_ref, o0_ref, o1_ref):
        dinv = _dinv_block(degp_ref)
        z = jnp.dot(x_ref[...], w_ref[...], preferred_element_type=jnp.float32)
        z = z * dinv
        o0_ref[...] = z[:, :doh]
        o1_ref[...] = z[:, doh:]

    return pl.pallas_call(
        body,
        grid=(GRID,),
        in_specs=[
            pl.BlockSpec((ROWB, d_in), lambda i: (i, 0)),
            pl.BlockSpec((d_in, do), lambda i: (0, 0)),
            pl.BlockSpec((NC, ROWB, 16), lambda i: (0, i, 0)),
        ],
        out_specs=outs,
        out_shape=shapes,
    )(x, W1, degp)


def _tc_layer(accp, zs0, zs1, degp, Wn, gs, cb):
    dp, do = Wn.shape
    dph = dp // 2
    doh, outs, shapes = _split_specs(do)

    def body(a_ref, zs0_ref, zs1_ref, degp_ref, w_ref, gs_ref, cb_ref,
             o0_ref, o1_ref):
        dinv = _dinv_block(degp_ref)
        acc = jnp.concatenate(
            [a_ref[0] + zs0_ref[...], a_ref[1] + zs1_ref[...]], axis=1)
        h = _lrelu(acc * dinv * gs_ref[...] + cb_ref[...])
        z = jnp.dot(h, w_ref[...], preferred_element_type=jnp.float32)
        z = z * dinv
        o0_ref[...] = z[:, :doh]
        o1_ref[...] = z[:, doh:]

    return pl.pallas_call(
        body,
        grid=(GRID,),
        in_specs=[
            pl.BlockSpec((NC, ROWB, dph), lambda i: (0, i, 0)),
            pl.BlockSpec((ROWB, dph), lambda i: (i, 0)),
            pl.BlockSpec((ROWB, dph), lambda i: (i, 0)),
            pl.BlockSpec((NC, ROWB, 16), lambda i: (0, i, 0)),
            pl.BlockSpec((dp, do), lambda i: (0, 0)),
            pl.BlockSpec((1, dp), lambda i: (0, 0)),
            pl.BlockSpec((1, dp), lambda i: (0, 0)),
        ],
        out_specs=outs,
        out_shape=shapes,
    )(accp, zs0, zs1, degp, Wn, gs, cb)


def _tc_pool_head(accp, zs0, zs1, degp, gs, cb, batch3, low,
                  Wg, bg, Wl, bl, Wc1, Wc2, bc, Wo, bo):
    dp = 2 * zs0.shape[1]
    dph = dp // 2

    def body(a_ref, zs0_ref, zs1_ref, degp_ref, gs_ref, cb_ref, b_ref,
             low_ref, wg_ref, bg_ref, wl_ref, bl_ref, wc1_ref, wc2_ref,
             bc_ref, wo_ref, bo_ref, q_ref, ps_ref, cnt_ref):
        i = pl.program_id(0)
        dinv = _dinv_block(degp_ref)
        acc = jnp.concatenate(
            [a_ref[0] + zs0_ref[...], a_ref[1] + zs1_ref[...]], axis=1)
        h = _lrelu(acc * dinv * gs_ref[...] + cb_ref[...])
        seg = b_ref[0]
        gid = lax.broadcasted_iota(jnp.int32, (G, 1), 0)
        oh = (seg == gid).astype(jnp.float32)
        ps = jnp.dot(oh, h, preferred_element_type=jnp.float32)
        cnt = jnp.sum(oh, axis=1, keepdims=True)
        cntb = jnp.broadcast_to(cnt, (G, dp))

        @pl.when(i == 0)
        def _():
            ps_ref[...] = ps
            cnt_ref[...] = cntb

        @pl.when(i > 0)
        def _():
            ps_ref[...] = ps_ref[...] + ps
            cnt_ref[...] = cnt_ref[...] + cntb

        @pl.when(i == GRID - 1)
        def _():
            xg = ps_ref[...] / jnp.maximum(cnt_ref[...], 1.0)
            xg = _lrelu(
                jnp.dot(xg, wg_ref[...], preferred_element_type=jnp.float32)
                + bg_ref[...])
            lo = _lrelu(jnp.dot(low_ref[...], wl_ref[...],
                                preferred_element_type=jnp.float32)
                        + bl_ref[...])
            comb = _lrelu(
                jnp.dot(xg, wc1_ref[...], preferred_element_type=jnp.float32)
                + jnp.dot(lo, wc2_ref[...], preferred_element_type=jnp.float32)
                + bc_ref[...])
            q_ref[...] = (jnp.dot(comb, wo_ref[...],
                                  preferred_element_type=jnp.float32)
                          + bo_ref[...])

    full = lambda a: pl.BlockSpec(a.shape, lambda i: tuple(0 for _ in a.shape))
    return pl.pallas_call(
        body,
        grid=(GRID,),
        in_specs=[
            pl.BlockSpec((NC, ROWB, dph), lambda i: (0, i, 0)),
            pl.BlockSpec((ROWB, dph), lambda i: (i, 0)),
            pl.BlockSpec((ROWB, dph), lambda i: (i, 0)),
            pl.BlockSpec((NC, ROWB, 16), lambda i: (0, i, 0)),
            pl.BlockSpec((1, dp), lambda i: (0, 0)),
            pl.BlockSpec((1, dp), lambda i: (0, 0)),
            pl.BlockSpec((1, 1, ROWB), lambda i: (i, 0, 0)),
            full(low), full(Wg), full(bg), full(Wl), full(bl),
            full(Wc1), full(Wc2), full(bc), full(Wo), full(bo),
        ],
        out_specs=pl.BlockSpec((G, 4), lambda i: (0, 0)),
        out_shape=jax.ShapeDtypeStruct((G, 4), jnp.float32),
        scratch_shapes=[
            pltpu.VMEM((G, dp), jnp.float32),
            pltpu.VMEM((G, dp), jnp.float32),
        ],
    )(accp, zs0, zs1, degp, gs, cb, batch3, low,
      Wg, bg, Wl, bl, Wc1, Wc2, bc, Wo, bo)


def kernel(x, edge_index, batch, low, W1, b1, g1, bt1, W2, b2, g2, bt2,
           W3, b3, g3, bt3, W4, b4, g4, bt4, Wg, bg, Wl, bl, Wc, bc, Wo, bo):
    batch3 = batch.reshape(GRID, 1, ROWB)
    pad = jnp.concatenate(
        [jnp.zeros((1, E_PAD - E), jnp.int32),
         jnp.full((1, E_PAD - E), N, jnp.int32)], axis=0)
    edge_index = jnp.concatenate([edge_index, pad], axis=1)

    sc = 1.0 / jnp.sqrt(jnp.float32(1.0 + 1e-5))
    gs1, cb1 = (g1 * sc).reshape(1, -1), (b1 * g1 * sc + bt1).reshape(1, -1)
    gs2, cb2 = (g2 * sc).reshape(1, -1), (b2 * g2 * sc + bt2).reshape(1, -1)
    gs3, cb3 = (g3 * sc).reshape(1, -1), (b3 * g3 * sc + bt3).reshape(1, -1)
    gs4, cb4 = (g4 * sc).reshape(1, -1), (b4 * g4 * sc + bt4).reshape(1, -1)

    degp = _sc_degree(edge_index)

    zs1a, zs1b = _tc_first(x, W1, degp)
    acc1 = _sc_scatter(edge_index, zs1a, zs1b, W1.shape[1] // 2, 10)
    zs2a, zs2b = _tc_layer(acc1, zs1a, zs1b, degp, W2, gs1, cb1)
    acc2 = _sc_scatter(edge_index, zs2a, zs2b, W2.shape[1] // 2, 10)
    zs3a, zs3b = _tc_layer(acc2, zs2a, zs2b, degp, W3, gs2, cb2)
    acc3 = _sc_scatter(edge_index, zs3a, zs3b, W3.shape[1] // 2, 10)
    zs4a, zs4b = _tc_layer(acc3, zs3a, zs3b, degp, W4, gs3, cb3)
    acc4 = _sc_scatter(edge_index, zs4a, zs4b, W4.shape[1] // 2, 10)

    q = _tc_pool_head(acc4, zs4a, zs4b, degp, gs4, cb4, batch3, low,
                      Wg, bg.reshape(1, -1), Wl, bl.reshape(1, -1),
                      Wc[:128], Wc[128:], bc.reshape(1, -1),
                      Wo, bo.reshape(1, -1))
    return q


# speedup vs baseline: 1.3098x; 1.3098x over previous
import jax
import jax.numpy as jnp
from jax import lax
from jax.experimental import pallas as pl
from jax.experimental.pallas import tpu as pltpu
from jax.experimental.pallas import tpu_sc as plsc

N = 10000
E = 320000
G = 64

NC = 2
NS = 16
CH = 80
EPT = E // NS
NROWS = N
ROWS = 624
TAIL = N - ROWS * NS

ROWB = 1000
GRID = N // ROWB


def _sc_mesh():
    return plsc.VectorSubcoreMesh(core_axis_name="c", subcore_axis_name="s")


def _zero_rows(ref, nrows, width):
    z16 = jnp.zeros((16,), jnp.float32)

    def zrow(r, carry):
        for j in range(width // 16):
            ref[r, pl.ds(j * 16, 16)] = z16
        return carry

    lax.fori_loop(0, nrows, zrow, 0)


def _init_table(bounce, acc_sh, s, width):
    _zero_rows(bounce, ROWS + TAIL, width)
    row0 = pl.multiple_of(s * ROWS, 8)
    pltpu.sync_copy(bounce.at[pl.ds(0, ROWS)], acc_sh.at[pl.ds(row0, ROWS)])

    @pl.when(s == NS - 1)
    def _():
        pltpu.sync_copy(bounce.at[pl.ds(ROWS, TAIL)],
                        acc_sh.at[pl.ds(NS * ROWS, TAIL)])


def _drain_table(bounce, acc_sh, out_h, c, s):
    row0 = pl.multiple_of(s * ROWS, 8)
    pltpu.sync_copy(acc_sh.at[pl.ds(row0, ROWS)], bounce.at[pl.ds(0, ROWS)])
    pltpu.sync_copy(bounce.at[pl.ds(0, ROWS)], out_h.at[c, pl.ds(row0, ROWS)])

    @pl.when(s == NS - 1)
    def _():
        pltpu.sync_copy(acc_sh.at[pl.ds(NS * ROWS, TAIL)],
                        bounce.at[pl.ds(ROWS, TAIL)])
        pltpu.sync_copy(bounce.at[pl.ds(ROWS, TAIL)],
                        out_h.at[c, pl.ds(NS * ROWS, TAIL)])


NBUF = 10

_SLICES = [(0, 80), (80, 80), (160, 80), (240, 80), (320, 80),
           (400, 80), (480, 80), (560, 64)]


def _init_table_slots(rows, acc_sh, s, width, nbuf):
    _zero_rows(rows.at[0], CH, width)
    row0 = pl.multiple_of(s * ROWS, 8)
    for off, sz in _SLICES:
        pltpu.sync_copy(rows.at[0, pl.ds(0, sz)],
                        acc_sh.at[pl.ds(row0 + off, sz)])

    @pl.when(s == NS - 1)
    def _():
        pltpu.sync_copy(rows.at[0, pl.ds(0, TAIL)],
                        acc_sh.at[pl.ds(NS * ROWS, TAIL)])


def _drain_table_slots(rows, acc_sh, out_h, c, s, nbuf, isem, ssem):
    row0 = pl.multiple_of(s * ROWS, 8)
    outd = {}
    for j, (off, sz) in enumerate(_SLICES):
        b = j % nbuf
        if b in outd:
            outd[b].wait()
        d1 = pltpu.async_copy(acc_sh.at[pl.ds(row0 + off, sz)],
                              rows.at[b, pl.ds(0, sz)], isem.at[b])
        d1.wait()
        outd[b] = pltpu.async_copy(rows.at[b, pl.ds(0, sz)],
                                   out_h.at[c, pl.ds(row0 + off, sz)],
                                   ssem.at[b])
    for b, d in outd.items():
        d.wait()

    @pl.when(s == NS - 1)
    def _():
        t0 = NS * ROWS
        d1 = pltpu.async_copy(acc_sh.at[pl.ds(t0, TAIL)],
                              rows.at[0, pl.ds(0, TAIL)], isem.at[0])
        d1.wait()
        d2 = pltpu.async_copy(rows.at[0, pl.ds(0, TAIL)],
                              out_h.at[c, pl.ds(t0, TAIL)], ssem.at[0])
        d2.wait()


def _sc_scatter(ei, zs0, zs1, doh, nbuf):
    ept = EPT
    group = nbuf * CH
    ngroups = ept // group
    assert ngroups * group == ept

    def body(ei_h, zs0_h, zs1_h, out_h, eidx, rows, acc_sh,
             isem, gsem, ssem):
        c = lax.axis_index("c")
        s = lax.axis_index("s")
        base = s * ept

        def phase(zs_h):
            idesc0 = []
            for b in range(nbuf):
                off = pl.multiple_of(base + b * CH, 16)
                idesc0.append(pltpu.async_copy(
                    ei_h.at[:, pl.ds(off, CH)], eidx.at[b], isem.at[b]))
            _init_table_slots(rows, acc_sh, s, doh, nbuf)
            gdesc0 = []
            for b in range(nbuf):
                idesc0[b].wait()
                gdesc0.append(pltpu.async_copy(
                    zs_h.at[eidx.at[b, 0]], rows.at[b], gsem.at[b]))
            plsc.subcore_barrier()
            for b in range(nbuf):
                gdesc0[b].wait()
                pltpu.async_copy(
                    rows.at[b], acc_sh.at[eidx.at[b, 1]], ssem.at[b],
                    add=True)

            def inner(g, pg):
                goff = pl.multiple_of(base + g * group, 16)
                idesc = []
                for b in range(nbuf):
                    idesc.append(pltpu.async_copy(
                        ei_h.at[:, pl.ds(goff + b * CH, CH)],
                        eidx.at[pg + b], isem.at[b]))
                gdesc = []
                for b in range(nbuf):
                    idesc[b].wait()
                    pltpu.make_async_copy(
                        zs_h.at[pl.ds(0, CH)], rows.at[b],
                        ssem.at[b]).wait()
                    gdesc.append(pltpu.async_copy(
                        zs_h.at[eidx.at[pg + b, 0]], rows.at[b], gsem.at[b]))
                for b in range(nbuf):
                    gdesc[b].wait()
                    pltpu.async_copy(
                        rows.at[b], acc_sh.at[eidx.at[pg + b, 1]], ssem.at[b],
                        add=True)

            def gbody(g, carry):
                @pl.when(g % 2 == 1)
                def _():
                    inner(g, nbuf)

                @pl.when(g % 2 == 0)
                def _():
                    inner(g, 0)

                return carry

            lax.fori_loop(1, ngroups, gbody, 0)
            for b in range(nbuf):
                pltpu.make_async_copy(
                    zs_h.at[pl.ds(0, CH)], rows.at[b], ssem.at[b]).wait()

        @pl.when(c == 0)
        def _():
            phase(zs0_h)

        @pl.when(c == 1)
        def _():
            phase(zs1_h)

        plsc.subcore_barrier()
        _drain_table_slots(rows, acc_sh, out_h, c, s, nbuf, isem, ssem)

    scratch = [
        pltpu.VMEM((2 * nbuf, 2, CH), jnp.int32),
        pltpu.VMEM((nbuf, CH, doh), jnp.float32),
        pltpu.VMEM_SHARED((NROWS, doh), jnp.float32),
        pltpu.SemaphoreType.DMA((nbuf,)),
        pltpu.SemaphoreType.DMA((nbuf,)),
        pltpu.SemaphoreType.DMA((nbuf,)),
    ]
    return pl.kernel(
        body,
        out_type=jax.ShapeDtypeStruct((NC, N, doh), jnp.float32),
        mesh=_sc_mesh(),
        scratch_types=scratch,
        compiler_params=pltpu.CompilerParams(use_tc_tiling_on_sc=False),
    )(ei, zs0, zs1)


def _sc_degree(ei):
    W = 16
    ept = EPT // NC
    group = 10 * CH
    nbuf = 10
    ngroups = ept // group

    def body(ei_h, out_h, didx, ones_v, bounce, acc_sh, isem, ssem):
        c = lax.axis_index("c")
        s = lax.axis_index("s")
        one16 = jnp.ones((16,), jnp.float32)

        def orow(r, carry):
            ones_v[r, pl.ds(0, 16)] = one16
            return carry

        lax.fori_loop(0, CH, orow, 0)
        _init_table(bounce, acc_sh, s, W)
        plsc.subcore_barrier()

        base = c * (E // NC) + s * ept

        def gbody(g, carry):
            goff = pl.multiple_of(base + g * group, 16)
            idesc = []
            for b in range(nbuf):
                idesc.append(pltpu.async_copy(
                    ei_h.at[1, pl.ds(goff + b * CH, CH)],
                    didx.at[b], isem.at[b]))
            sdesc = []
            for b in range(nbuf):
                idesc[b].wait()
                sdesc.append(pltpu.async_copy(
                    ones_v, acc_sh.at[didx.at[b]], ssem.at[b], add=True))
            for b in range(nbuf):
                sdesc[b].wait()
            return carry

        lax.fori_loop(0, ngroups, gbody, 0)
        plsc.subcore_barrier()
        _drain_table(bounce, acc_sh, out_h, c, s)

    scratch = [
        pltpu.VMEM((10, CH), jnp.int32),
        pltpu.VMEM((CH, W), jnp.float32),
        pltpu.VMEM((ROWS + TAIL, W), jnp.float32),
        pltpu.VMEM_SHARED((NROWS, W), jnp.float32),
        pltpu.SemaphoreType.DMA((10,)),
        pltpu.SemaphoreType.DMA((10,)),
    ]
    return pl.kernel(
        body,
        out_type=jax.ShapeDtypeStruct((NC, N, W), jnp.float32),
        mesh=_sc_mesh(),
        scratch_types=scratch,
        compiler_params=pltpu.CompilerParams(use_tc_tiling_on_sc=False),
    )(ei)


def _lrelu(x):
    return jnp.where(x > 0, x, 0.01 * x)


def _dinv_block(degp_ref):
    deg = degp_ref[0, :, 0:1] + degp_ref[1, :, 0:1] + 1.0
    return lax.rsqrt(deg)


def _split_specs(do):
    doh = do // 2
    outs = [pl.BlockSpec((ROWB, doh), lambda i: (i, 0)) for _ in range(2)]
    shapes = [jax.ShapeDtypeStruct((N, doh), jnp.float32) for _ in range(2)]
    return doh, outs, shapes


def _tc_first(x, W1, degp):
    d_in, do = W1.shape
    doh, outs, shapes = _split_specs(do)

    def body(x_ref, w_ref, degp_ref, o0_ref, o1_ref):
        dinv = _dinv_block(degp_ref)
        z = jnp.dot(x_ref[...], w_ref[...], preferred_element_type=jnp.float32)
        z = z * dinv
        o0_ref[...] = z[:, :doh]
        o1_ref[...] = z[:, doh:]

    return pl.pallas_call(
        body,
        grid=(GRID,),
        in_specs=[
            pl.BlockSpec((ROWB, d_in), lambda i: (i, 0)),
            pl.BlockSpec((d_in, do), lambda i: (0, 0)),
            pl.BlockSpec((NC, ROWB, 16), lambda i: (0, i, 0)),
        ],
        out_specs=outs,
        out_shape=shapes,
    )(x, W1, degp)


def _tc_layer(accp, zs0, zs1, degp, Wn, gs, cb):
    dp, do = Wn.shape
    dph = dp // 2
    doh, outs, shapes = _split_specs(do)

    def body(a_ref, zs0_ref, zs1_ref, degp_ref, w_ref, gs_ref, cb_ref,
             o0_ref, o1_ref):
        dinv = _dinv_block(degp_ref)
        acc = jnp.concatenate(
            [a_ref[0] + zs0_ref[...], a_ref[1] + zs1_ref[...]], axis=1)
        h = _lrelu(acc * dinv * gs_ref[...] + cb_ref[...])
        z = jnp.dot(h, w_ref[...], preferred_element_type=jnp.float32)
        z = z * dinv
        o0_ref[...] = z[:, :doh]
        o1_ref[...] = z[:, doh:]

    return pl.pallas_call(
        body,
        grid=(GRID,),
        in_specs=[
            pl.BlockSpec((NC, ROWB, dph), lambda i: (0, i, 0)),
            pl.BlockSpec((ROWB, dph), lambda i: (i, 0)),
            pl.BlockSpec((ROWB, dph), lambda i: (i, 0)),
            pl.BlockSpec((NC, ROWB, 16), lambda i: (0, i, 0)),
            pl.BlockSpec((dp, do), lambda i: (0, 0)),
            pl.BlockSpec((1, dp), lambda i: (0, 0)),
            pl.BlockSpec((1, dp), lambda i: (0, 0)),
        ],
        out_specs=outs,
        out_shape=shapes,
    )(accp, zs0, zs1, degp, Wn, gs, cb)


def _tc_pool_head(accp, zs0, zs1, degp, gs, cb, batch3, low,
                  Wg, bg, Wl, bl, Wc1, Wc2, bc, Wo, bo):
    dp = 2 * zs0.shape[1]
    dph = dp // 2

    def body(a_ref, zs0_ref, zs1_ref, degp_ref, gs_ref, cb_ref, b_ref,
             low_ref, wg_ref, bg_ref, wl_ref, bl_ref, wc1_ref, wc2_ref,
             bc_ref, wo_ref, bo_ref, q_ref, ps_ref, cnt_ref):
        i = pl.program_id(0)
        dinv = _dinv_block(degp_ref)
        acc = jnp.concatenate(
            [a_ref[0] + zs0_ref[...], a_ref[1] + zs1_ref[...]], axis=1)
        h = _lrelu(acc * dinv * gs_ref[...] + cb_ref[...])
        seg = b_ref[0]
        gid = lax.broadcasted_iota(jnp.int32, (G, 1), 0)
        oh = (seg == gid).astype(jnp.float32)
        ps = jnp.dot(oh, h, preferred_element_type=jnp.float32)
        cnt = jnp.sum(oh, axis=1, keepdims=True)
        cntb = jnp.broadcast_to(cnt, (G, dp))

        @pl.when(i == 0)
        def _():
            ps_ref[...] = ps
            cnt_ref[...] = cntb

        @pl.when(i > 0)
        def _():
            ps_ref[...] = ps_ref[...] + ps
            cnt_ref[...] = cnt_ref[...] + cntb

        @pl.when(i == GRID - 1)
        def _():
            xg = ps_ref[...] / jnp.maximum(cnt_ref[...], 1.0)
            xg = _lrelu(
                jnp.dot(xg, wg_ref[...], preferred_element_type=jnp.float32)
                + bg_ref[...])
            lo = _lrelu(jnp.dot(low_ref[...], wl_ref[...],
                                preferred_element_type=jnp.float32)
                        + bl_ref[...])
            comb = _lrelu(
                jnp.dot(xg, wc1_ref[...], preferred_element_type=jnp.float32)
                + jnp.dot(lo, wc2_ref[...], preferred_element_type=jnp.float32)
                + bc_ref[...])
            q_ref[...] = (jnp.dot(comb, wo_ref[...],
                                  preferred_element_type=jnp.float32)
                          + bo_ref[...])

    full = lambda a: pl.BlockSpec(a.shape, lambda i: tuple(0 for _ in a.shape))
    return pl.pallas_call(
        body,
        grid=(GRID,),
        in_specs=[
            pl.BlockSpec((NC, ROWB, dph), lambda i: (0, i, 0)),
            pl.BlockSpec((ROWB, dph), lambda i: (i, 0)),
            pl.BlockSpec((ROWB, dph), lambda i: (i, 0)),
            pl.BlockSpec((NC, ROWB, 16), lambda i: (0, i, 0)),
            pl.BlockSpec((1, dp), lambda i: (0, 0)),
            pl.BlockSpec((1, dp), lambda i: (0, 0)),
            pl.BlockSpec((1, 1, ROWB), lambda i: (i, 0, 0)),
            full(low), full(Wg), full(bg), full(Wl), full(bl),
            full(Wc1), full(Wc2), full(bc), full(Wo), full(bo),
        ],
        out_specs=pl.BlockSpec((G, 4), lambda i: (0, 0)),
        out_shape=jax.ShapeDtypeStruct((G, 4), jnp.float32),
        scratch_shapes=[
            pltpu.VMEM((G, dp), jnp.float32),
            pltpu.VMEM((G, dp), jnp.float32),
        ],
    )(accp, zs0, zs1, degp, gs, cb, batch3, low,
      Wg, bg, Wl, bl, Wc1, Wc2, bc, Wo, bo)


def kernel(x, edge_index, batch, low, W1, b1, g1, bt1, W2, b2, g2, bt2,
           W3, b3, g3, bt3, W4, b4, g4, bt4, Wg, bg, Wl, bl, Wc, bc, Wo, bo):
    batch3 = batch.reshape(GRID, 1, ROWB)

    sc = 1.0 / jnp.sqrt(jnp.float32(1.0 + 1e-5))
    gs1, cb1 = (g1 * sc).reshape(1, -1), (b1 * g1 * sc + bt1).reshape(1, -1)
    gs2, cb2 = (g2 * sc).reshape(1, -1), (b2 * g2 * sc + bt2).reshape(1, -1)
    gs3, cb3 = (g3 * sc).reshape(1, -1), (b3 * g3 * sc + bt3).reshape(1, -1)
    gs4, cb4 = (g4 * sc).reshape(1, -1), (b4 * g4 * sc + bt4).reshape(1, -1)

    degp = _sc_degree(edge_index)

    zs1a, zs1b = _tc_first(x, W1, degp)
    acc1 = _sc_scatter(edge_index, zs1a, zs1b, W1.shape[1] // 2, 10)
    zs2a, zs2b = _tc_layer(acc1, zs1a, zs1b, degp, W2, gs1, cb1)
    acc2 = _sc_scatter(edge_index, zs2a, zs2b, W2.shape[1] // 2, 10)
    zs3a, zs3b = _tc_layer(acc2, zs2a, zs2b, degp, W3, gs2, cb2)
    acc3 = _sc_scatter(edge_index, zs3a, zs3b, W3.shape[1] // 2, 10)
    zs4a, zs4b = _tc_layer(acc3, zs3a, zs3b, degp, W4, gs3, cb3)
    acc4 = _sc_scatter(edge_index, zs4a, zs4b, W4.shape[1] // 2, 10)

    q = _tc_pool_head(acc4, zs4a, zs4b, degp, gs4, cb4, batch3, low,
                      Wg, bg.reshape(1, -1), Wl, bl.reshape(1, -1),
                      Wc[:128], Wc[128:], bc.reshape(1, -1),
                      Wo, bo.reshape(1, -1))
    return q
